# baseline (speedup 1.0000x reference)
import functools

import jax
import jax.numpy as jnp
from jax.experimental import pallas as pl
from jax.experimental.pallas import tpu as pltpu

_ROWS = 512
_NBUF = 6
_LOOKAHEAD = 5


def _chunk_copy(x_hbm, xbuf, sems, c):
    return pltpu.make_async_copy(
        x_hbm.at[pl.ds(c * _ROWS, _ROWS), :],
        xbuf.at[c % _NBUF],
        sems.at[c % _NBUF],
    )


def _out_copies(obuf_l, obuf_w, obuf_i, logits_hbm, w_hbm, idx_hbm,
                osems, c):
    rows = pl.ds(c * _ROWS, _ROWS)
    slot = c % 2
    return (
        pltpu.make_async_copy(obuf_l.at[slot], logits_hbm.at[rows, :],
                              osems.at[slot, 0]),
        pltpu.make_async_copy(obuf_w.at[slot], w_hbm.at[rows, :],
                              osems.at[slot, 1]),
        pltpu.make_async_copy(obuf_i.at[slot], idx_hbm.at[rows, :],
                              osems.at[slot, 2]),
    )


def _router_body(x_hbm, wt_ref, logits_hbm, w_hbm, idx_hbm,
                 xbuf, sems, obuf_l, obuf_w, obuf_i, osems,
                 *, nchunks, n_experts):
    c = pl.program_id(0)
    slot = c % 2

    @pl.when(c == 0)
    def _prologue():
        for k in range(_LOOKAHEAD):
            _chunk_copy(x_hbm, xbuf, sems, k).start()

    @pl.when(c + _LOOKAHEAD < nchunks)
    def _prefetch():
        _chunk_copy(x_hbm, xbuf, sems, c + _LOOKAHEAD).start()

    @pl.when(c >= 2)
    def _wait_prev_out():
        for cp in _out_copies(obuf_l, obuf_w, obuf_i, logits_hbm, w_hbm,
                              idx_hbm, osems, c - 2):
            cp.wait()

    _chunk_copy(x_hbm, xbuf, sems, c).wait()
    l = jnp.dot(xbuf[c % _NBUF], wt_ref[:, :],
                preferred_element_type=jnp.float32)
    m = jnp.max(l, axis=1, keepdims=True)
    s = jnp.sum(jnp.exp(l - m), axis=1, keepdims=True)
    iota = jax.lax.broadcasted_iota(jnp.int32, l.shape, 1)
    idx = jnp.min(jnp.where(l == m, iota, n_experts), axis=1, keepdims=True)
    obuf_l[slot] = l
    obuf_w[slot] = 1.0 / s
    obuf_i[slot] = idx
    for cp in _out_copies(obuf_l, obuf_w, obuf_i, logits_hbm, w_hbm,
                          idx_hbm, osems, c):
        cp.start()

    @pl.when(c == nchunks - 1)
    def _epilogue():
        for last in (nchunks - 2, nchunks - 1):
            for cp in _out_copies(obuf_l, obuf_w, obuf_i, logits_hbm,
                                  w_hbm, idx_hbm, osems, last):
                cp.wait()


def kernel(hidden_states, W):
    b, s, h = hidden_states.shape
    e = W.shape[0]
    n = b * s
    x = hidden_states.reshape(n, h)
    wt = W.T
    nchunks = n // _ROWS

    logits, weights, indices = pl.pallas_call(
        functools.partial(_router_body, nchunks=nchunks, n_experts=e),
        grid=(nchunks,),
        in_specs=[
            pl.BlockSpec(memory_space=pl.ANY),
            pl.BlockSpec((h, e), lambda i: (0, 0)),
        ],
        out_specs=[
            pl.BlockSpec(memory_space=pl.ANY),
            pl.BlockSpec(memory_space=pl.ANY),
            pl.BlockSpec(memory_space=pl.ANY),
        ],
        out_shape=[
            jax.ShapeDtypeStruct((n, e), jnp.float32),
            jax.ShapeDtypeStruct((n, 1), jnp.float32),
            jax.ShapeDtypeStruct((n, 1), jnp.int32),
        ],
        scratch_shapes=[
            pltpu.VMEM((_NBUF, _ROWS, h), jnp.float32),
            pltpu.SemaphoreType.DMA((_NBUF,)),
            pltpu.VMEM((2, _ROWS, e), jnp.float32),
            pltpu.VMEM((2, _ROWS, 1), jnp.float32),
            pltpu.VMEM((2, _ROWS, 1), jnp.int32),
            pltpu.SemaphoreType.DMA((2, 3)),
        ],
        compiler_params=pltpu.CompilerParams(
            dimension_semantics=("arbitrary",),
        ),
    )(x, wt)

    return (weights.reshape(b, s, 1),
            indices.reshape(b, s, 1),
            logits.reshape(b, s, e))


# speedup vs baseline: 1.0104x; 1.0104x over previous
import functools

import jax
import jax.numpy as jnp
from jax.experimental import pallas as pl
from jax.experimental.pallas import tpu as pltpu

_ROWS = 256
_NBUF = 8
_LOOKAHEAD = 7


def _chunk_copy(x_hbm, xbuf, sems, c):
    return pltpu.make_async_copy(
        x_hbm.at[pl.ds(c * _ROWS, _ROWS), :],
        xbuf.at[c % _NBUF],
        sems.at[c % _NBUF],
    )


def _out_copies(obuf_l, obuf_w, obuf_i, logits_hbm, w_hbm, idx_hbm,
                osems, c):
    rows = pl.ds(c * _ROWS, _ROWS)
    slot = c % 2
    return (
        pltpu.make_async_copy(obuf_l.at[slot], logits_hbm.at[rows, :],
                              osems.at[slot, 0]),
        pltpu.make_async_copy(obuf_w.at[slot], w_hbm.at[rows, :],
                              osems.at[slot, 1]),
        pltpu.make_async_copy(obuf_i.at[slot], idx_hbm.at[rows, :],
                              osems.at[slot, 2]),
    )


def _router_body(x_hbm, wt_ref, logits_hbm, w_hbm, idx_hbm,
                 xbuf, sems, obuf_l, obuf_w, obuf_i, osems,
                 *, nchunks, n_experts):
    c = pl.program_id(0)
    slot = c % 2

    @pl.when(c == 0)
    def _prologue():
        for k in range(_LOOKAHEAD):
            _chunk_copy(x_hbm, xbuf, sems, k).start()

    @pl.when(c + _LOOKAHEAD < nchunks)
    def _prefetch():
        _chunk_copy(x_hbm, xbuf, sems, c + _LOOKAHEAD).start()

    @pl.when(c >= 2)
    def _wait_prev_out():
        for cp in _out_copies(obuf_l, obuf_w, obuf_i, logits_hbm, w_hbm,
                              idx_hbm, osems, c - 2):
            cp.wait()

    _chunk_copy(x_hbm, xbuf, sems, c).wait()
    l = jnp.dot(xbuf[c % _NBUF], wt_ref[:, :],
                preferred_element_type=jnp.float32)
    m = jnp.max(l, axis=1, keepdims=True)
    s = jnp.sum(jnp.exp(l - m), axis=1, keepdims=True)
    iota = jax.lax.broadcasted_iota(jnp.int32, l.shape, 1)
    idx = jnp.min(jnp.where(l == m, iota, n_experts), axis=1, keepdims=True)
    obuf_l[slot] = l
    obuf_w[slot] = 1.0 / s
    obuf_i[slot] = idx
    for cp in _out_copies(obuf_l, obuf_w, obuf_i, logits_hbm, w_hbm,
                          idx_hbm, osems, c):
        cp.start()

    @pl.when(c == nchunks - 1)
    def _epilogue():
        for last in (nchunks - 2, nchunks - 1):
            for cp in _out_copies(obuf_l, obuf_w, obuf_i, logits_hbm,
                                  w_hbm, idx_hbm, osems, last):
                cp.wait()


def kernel(hidden_states, W):
    b, s, h = hidden_states.shape
    e = W.shape[0]
    n = b * s
    x = hidden_states.reshape(n, h)
    wt = W.T
    nchunks = n // _ROWS

    logits, weights, indices = pl.pallas_call(
        functools.partial(_router_body, nchunks=nchunks, n_experts=e),
        grid=(nchunks,),
        in_specs=[
            pl.BlockSpec(memory_space=pl.ANY),
            pl.BlockSpec((h, e), lambda i: (0, 0)),
        ],
        out_specs=[
            pl.BlockSpec(memory_space=pl.ANY),
            pl.BlockSpec(memory_space=pl.ANY),
            pl.BlockSpec(memory_space=pl.ANY),
        ],
        out_shape=[
            jax.ShapeDtypeStruct((n, e), jnp.float32),
            jax.ShapeDtypeStruct((n, 1), jnp.float32),
            jax.ShapeDtypeStruct((n, 1), jnp.int32),
        ],
        scratch_shapes=[
            pltpu.VMEM((_NBUF, _ROWS, h), jnp.float32),
            pltpu.SemaphoreType.DMA((_NBUF,)),
            pltpu.VMEM((2, _ROWS, e), jnp.float32),
            pltpu.VMEM((2, _ROWS, 1), jnp.float32),
            pltpu.VMEM((2, _ROWS, 1), jnp.int32),
            pltpu.SemaphoreType.DMA((2, 3)),
        ],
        compiler_params=pltpu.CompilerParams(
            dimension_semantics=("arbitrary",),
        ),
    )(x, wt)

    return (weights.reshape(b, s, 1),
            indices.reshape(b, s, 1),
            logits.reshape(b, s, e))
